# baseline (speedup 1.0000x reference)
import jax
import jax.numpy as jnp
from jax.experimental import pallas as pl
from jax.experimental.pallas import tpu as pltpu

F_IN = 480
F_TGT = 512
B = 4096
C = 8
R = 4
NBT = B // 128
NFT = F_TGT // 128
SLAB = NBT * R * 128
NSTEPS = C * NFT


def _tc_body(perm_ref, x_ref, y_ref, buf, yblk, gsem0, gsem1, wsem0, wsem1):
    gsem = (gsem0, gsem1)
    wsem = (wsem0, wsem1)
    p = pl.program_id(0)
    npairs = pl.num_programs(0)

    def start_gather(sl, step):
        cc = step // NFT
        ftc = step % NFT
        for fl in range(128):
            pv = perm_ref[ftc * 128 + fl]

            @pl.when(pv < F_IN)
            def _():
                pltpu.make_async_copy(
                    x_ref.at[pl.ds((pv * C + cc) * 128, 128)],
                    buf.at[sl, fl],
                    gsem[sl],
                ).start()

    def finish_gather(sl, step):
        cc = step // NFT
        ftc = step % NFT
        for fl in range(128):
            pv = perm_ref[ftc * 128 + fl]

            @pl.when(pv < F_IN)
            def _():
                pltpu.make_async_copy(
                    x_ref.at[pl.ds((pv * C + cc) * 128, 128)],
                    buf.at[sl, fl],
                    gsem[sl],
                ).wait()

            @pl.when(pv >= F_IN)
            def _():
                buf[sl, fl, :, :] = jnp.zeros((NBT * R, 128), jnp.float32)

    def compute_and_write(sl, step):
        ws = sl % 2
        for bt in range(NBT):
            for r in range(R):
                t = buf[sl, :, bt * R + r, :]
                yblk[ws, pl.ds(bt * 128, 128), pl.ds(r * 128, 128)] = t.T
            if bt % 8 == 7:
                g = bt // 8
                pltpu.make_async_copy(
                    yblk.at[ws, pl.ds(g * 1024, 1024)],
                    y_ref.at[pl.ds(g * 1024, 1024), step, :],
                    wsem[ws],
                ).start()

    def drain_write(sl):
        pltpu.make_async_copy(yblk.at[sl], y_ref.at[:, 0, :], wsem[sl]).wait()

    a = p * 2

    @pl.when(p == 0)
    def _():
        start_gather(0, a)

    start_gather(1, a + 1)

    @pl.when(p >= 1)
    def _():
        drain_write(0)

    finish_gather(0, a)
    compute_and_write(0, a)

    @pl.when(p + 1 < npairs)
    def _():
        start_gather(0, a + 2)

    @pl.when(p >= 1)
    def _():
        drain_write(1)

    finish_gather(1, a + 1)
    compute_and_write(1, a + 1)

    @pl.when(p == npairs - 1)
    def _():
        drain_write(0)
        drain_write(1)


def kernel(ll, perm):
    b, f, c, r = ll.shape
    f_tgt = perm.shape[0]

    x2 = (
        ll.reshape(NBT, 128, f, c, r)
        .transpose(2, 3, 0, 4, 1)
        .reshape(f * c * NBT * r, 128)
    )

    y3 = pl.pallas_call(
        _tc_body,
        grid=(NSTEPS // 2,),
        in_specs=[
            pl.BlockSpec(memory_space=pltpu.SMEM),
            pl.BlockSpec(memory_space=pl.ANY),
        ],
        out_specs=pl.BlockSpec(memory_space=pl.ANY),
        out_shape=jax.ShapeDtypeStruct((b, NSTEPS, 512), jnp.float32),
        scratch_shapes=[
            pltpu.VMEM((2, 128, NBT * R, 128), jnp.float32),
            pltpu.VMEM((2, B, 512), jnp.float32),
            pltpu.SemaphoreType.DMA,
            pltpu.SemaphoreType.DMA,
            pltpu.SemaphoreType.DMA,
            pltpu.SemaphoreType.DMA,
        ],
        compiler_params=pltpu.CompilerParams(
            dimension_semantics=("arbitrary",),
            vmem_limit_bytes=110 * 1024 * 1024,
        ),
    )(perm, x2)

    return (
        y3.reshape(b, c, NFT, r, 128)
        .transpose(0, 2, 4, 1, 3)
        .reshape(b, f_tgt, c, r)
    )


# speedup vs baseline: 1.0182x; 1.0182x over previous
import jax
import jax.numpy as jnp
from jax.experimental import pallas as pl
from jax.experimental.pallas import tpu as pltpu

F_IN = 480
F_TGT = 512
B = 4096
C = 8
R = 4
NBT = B // 128
NFT = F_TGT // 128
SLAB = NBT * R * 128
NSTEPS = C * NFT


def _tc_body(perm_ref, x_ref, y_ref, buf, yblk,
             gsem0, gsem1, gsem2, gsem3, wsem0, wsem1):
    gsem = (gsem0, gsem1, gsem2, gsem3)
    wsem = (wsem0, wsem1)
    p = pl.program_id(0)
    npairs = pl.num_programs(0)

    def start_gather(sl, step):
        cc = step // NFT
        ftc = step % NFT
        for fl in range(128):
            pv = perm_ref[ftc * 128 + fl]

            @pl.when(pv < F_IN)
            def _():
                pltpu.make_async_copy(
                    x_ref.at[pl.ds((pv * C + cc) * 128, 128)],
                    buf.at[sl, fl],
                    gsem[sl],
                ).start()

    def finish_gather(sl, step):
        cc = step // NFT
        ftc = step % NFT
        for fl in range(128):
            pv = perm_ref[ftc * 128 + fl]

            @pl.when(pv < F_IN)
            def _():
                pltpu.make_async_copy(
                    x_ref.at[pl.ds((pv * C + cc) * 128, 128)],
                    buf.at[sl, fl],
                    gsem[sl],
                ).wait()

            @pl.when(pv >= F_IN)
            def _():
                buf[sl, fl, :, :] = jnp.zeros((NBT * R, 128), jnp.float32)

    def compute_and_write(sl, step):
        ws = sl % 2
        for bt in range(NBT):
            for r in range(R):
                t = buf[sl, :, bt * R + r, :]
                yblk[ws, pl.ds(bt * 128, 128), pl.ds(r * 128, 128)] = t.T
        pltpu.make_async_copy(yblk.at[ws], y_ref.at[:, step, :], wsem[ws]).start()

    def drain_write(sl):
        pltpu.make_async_copy(yblk.at[sl], y_ref.at[:, 0, :], wsem[sl]).wait()

    a = p * 4

    @pl.when(p == 0)
    def _():
        start_gather(0, 0)
        start_gather(1, 1)

    for i in range(4):
        @pl.when(a + i + 2 < NSTEPS)
        def _(i=i):
            start_gather((i + 2) % 4, a + i + 2)

        if i < 2:
            @pl.when(p >= 1)
            def _(i=i):
                drain_write(i % 2)
        else:
            drain_write(i % 2)
        finish_gather(i, a + i)
        compute_and_write(i, a + i)

    @pl.when(p == npairs - 1)
    def _():
        drain_write(0)
        drain_write(1)


def kernel(ll, perm):
    b, f, c, r = ll.shape
    f_tgt = perm.shape[0]

    x2 = (
        ll.reshape(NBT, 128, f, c, r)
        .transpose(2, 3, 0, 4, 1)
        .reshape(f * c * NBT * r, 128)
    )

    y3 = pl.pallas_call(
        _tc_body,
        grid=(NSTEPS // 4,),
        in_specs=[
            pl.BlockSpec(memory_space=pltpu.SMEM),
            pl.BlockSpec(memory_space=pl.ANY),
        ],
        out_specs=pl.BlockSpec(memory_space=pl.ANY),
        out_shape=jax.ShapeDtypeStruct((b, NSTEPS, 512), jnp.float32),
        scratch_shapes=[
            pltpu.VMEM((4, 128, NBT * R, 128), jnp.float32),
            pltpu.VMEM((2, B, 512), jnp.float32),
            pltpu.SemaphoreType.DMA,
            pltpu.SemaphoreType.DMA,
            pltpu.SemaphoreType.DMA,
            pltpu.SemaphoreType.DMA,
            pltpu.SemaphoreType.DMA,
            pltpu.SemaphoreType.DMA,
        ],
        compiler_params=pltpu.CompilerParams(
            dimension_semantics=("arbitrary",),
            vmem_limit_bytes=110 * 1024 * 1024,
        ),
    )(perm, x2)

    return (
        y3.reshape(b, c, NFT, r, 128)
        .transpose(0, 2, 4, 1, 3)
        .reshape(b, f_tgt, c, r)
    )
